# Initial kernel scaffold; baseline (speedup 1.0000x reference)
import jax
import jax.numpy as jnp
from jax.experimental import pallas as pl


def kernel(h0, h1, edge_index0, edge_index1, W_conv_td, b_conv_td, W_fus_td, b_fus_td, conv_w_td, td_w, Wcat_td, bcat_td, ln_g_td, ln_b_td, W_conv_bu, b_conv_bu, W_fus_bu, b_fus_bu, conv_w_bu, bu_w, Wcat_bu, bcat_bu, ln_g_bu, ln_b_bu):
    raise NotImplementedError("write your pallas kernel here")


# speedup vs baseline: 7.2903x; 7.2903x over previous
import functools

import jax
import jax.numpy as jnp
from jax import lax
from jax.experimental import pallas as pl
from jax.experimental.pallas import tpu as pltpu
from jax.experimental.pallas import tpu_sc as plsc

N0, E0 = 10000, 160000
N1, E1 = 160000, 480000
D = 128

NC, NS, L = 2, 16, 16
N0P = 10240
N1P = 163840
E0P = 163840
E1P = 491520
C0 = E0P // (NC * NS)
C1 = E1P // (NC * NS)
NB = 8192
NBK = N1P // NB
CE1 = E1P // NS
CROWS = 264
BLK = 1024


_SC_PARAMS = pltpu.CompilerParams(needs_layout_passes=False)


def _mesh():
  return plsc.VectorSubcoreMesh(core_axis_name="c", subcore_axis_name="s",
                                num_cores=NC, num_subcores=NS)


def _pass_a0_body(src0, dst0, h0p, h1p, zflat, zrows, ones,
                  histA, histB, tdp, bu,
                  hA, hB, td,
                  eidx, onev, rbuf, gbuf, sem):
  c = lax.axis_index("c")
  s = lax.axis_index("s")
  wid = c * NS + s

  pltpu.sync_copy(zflat.at[pl.ds(0, 640)], hA.at[pl.ds(s * 640, 640)])
  pltpu.sync_copy(zflat.at[pl.ds(0, 640)], hB.at[pl.ds(s * 640, 640)])
  pltpu.sync_copy(zrows, td.at[pl.ds(s * 640, 640), :])
  pltpu.sync_copy(ones, onev.at[0])
  plsc.subcore_barrier()

  def body0(j, _):
    base = wid * C0 + j * 128
    pltpu.sync_copy(src0.at[pl.ds(base, 128)], eidx.at[0])
    pltpu.sync_copy(dst0.at[pl.ds(base, 128)], eidx.at[1])
    pltpu.sync_copy(onev.at[0], hA.at[eidx.at[0]], add=True)
    pltpu.sync_copy(onev.at[0], hB.at[eidx.at[1]], add=True)
    pltpu.sync_copy(h1p.at[pl.ds(base, 128), :], rbuf)
    pltpu.sync_copy(rbuf, td.at[eidx.at[0]], add=True)
    pltpu.async_copy(h0p.at[eidx.at[0]], gbuf, sem).wait()
    pltpu.sync_copy(gbuf, bu.at[pl.ds(base, 128), :])
    return _

  lax.fori_loop(0, C0 // 128, body0, 0)
  plsc.subcore_barrier()

  pltpu.sync_copy(hA.at[pl.ds(s * 640, 640)], histA.at[c, pl.ds(s * 640, 640)])
  pltpu.sync_copy(hB.at[pl.ds(s * 640, 640)], histB.at[c, pl.ds(s * 640, 640)])
  pltpu.sync_copy(td.at[pl.ds(s * 640, 640), :],
                  tdp.at[c, pl.ds(s * 640, 640), :])


def _pass_a1_body(src1, dst1, zflat, ones,
                  histC, histD,
                  hC, hD,
                  eidx, onev):
  c = lax.axis_index("c")
  s = lax.axis_index("s")
  wid = c * NS + s

  pltpu.sync_copy(zflat, hC.at[pl.ds(s * 10240, 8192)])
  pltpu.sync_copy(zflat.at[pl.ds(0, 2048)],
                  hC.at[pl.ds(s * 10240 + 8192, 2048)])
  pltpu.sync_copy(zflat, hD.at[pl.ds(s * 10240, 8192)])
  pltpu.sync_copy(zflat.at[pl.ds(0, 2048)],
                  hD.at[pl.ds(s * 10240 + 8192, 2048)])
  pltpu.sync_copy(ones, onev.at[0])
  plsc.subcore_barrier()

  def body1(j, _):
    base = wid * C1 + j * 128
    pltpu.sync_copy(src1.at[pl.ds(base, 128)], eidx.at[0])
    pltpu.sync_copy(dst1.at[pl.ds(base, 128)], eidx.at[1])
    pltpu.sync_copy(onev.at[0], hC.at[eidx.at[0]], add=True)
    pltpu.sync_copy(onev.at[0], hD.at[eidx.at[1]], add=True)
    return _

  lax.fori_loop(0, C1 // 128, body1, 0)
  plsc.subcore_barrier()

  pltpu.sync_copy(hC.at[pl.ds(s * 10240, 10240)],
                  histC.at[c, pl.ds(s * 10240, 10240)])
  pltpu.sync_copy(hD.at[pl.ds(s * 10240, 10240)],
                  histD.at[c, pl.ds(s * 10240, 10240)])


def _run_pass_a(src0, dst0, src1, dst1, h0p, h1p):
  zflat = jnp.zeros((8192,), jnp.float32)
  zrows = jnp.zeros((640, D), jnp.float32)
  ones = jnp.ones((128,), jnp.float32)
  out0 = [
      jax.ShapeDtypeStruct((NC, N0P), jnp.float32),
      jax.ShapeDtypeStruct((NC, N0P), jnp.float32),
      jax.ShapeDtypeStruct((NC, N0P, D), jnp.float32),
      jax.ShapeDtypeStruct((N1P, D), jnp.float32),
  ]
  scratch0 = [
      pltpu.VMEM_SHARED((N0P,), jnp.float32),
      pltpu.VMEM_SHARED((N0P,), jnp.float32),
      pltpu.VMEM_SHARED((N0P, D), jnp.float32),
      pltpu.VMEM((2, 128), jnp.int32),
      pltpu.VMEM((1, 128), jnp.float32),
      pltpu.VMEM((128, D), jnp.float32),
      pltpu.VMEM((128, D), jnp.float32),
      pltpu.SemaphoreType.DMA,
  ]
  f0 = pl.kernel(_pass_a0_body, out_type=out0, mesh=_mesh(),
                 compiler_params=_SC_PARAMS, scratch_types=scratch0)
  histA, histB, tdp, bu = f0(src0, dst0, h0p, h1p, zflat, zrows, ones)

  out1 = [
      jax.ShapeDtypeStruct((NC, N1P), jnp.float32),
      jax.ShapeDtypeStruct((NC, N1P), jnp.float32),
  ]
  scratch1 = [
      pltpu.VMEM_SHARED((N1P,), jnp.float32),
      pltpu.VMEM_SHARED((N1P,), jnp.float32),
      pltpu.VMEM((2, 128), jnp.int32),
      pltpu.VMEM((1, 128), jnp.float32),
  ]
  f1 = pl.kernel(_pass_a1_body, out_type=out1, mesh=_mesh(),
                 compiler_params=_SC_PARAMS, scratch_types=scratch1)
  histC, histD = f1(src1, dst1, zflat, ones)
  return histA, histB, histC, histD, tdp, bu


def _agg0_one(table, out, src0, dst0, acc, eidx, gbuf, sem, s):
  pltpu.sync_copy(table.at[pl.ds(s * 640, 640), :], acc.at[pl.ds(s * 640, 640), :])
  plsc.subcore_barrier()

  def body(j, _):
    base = s * (E0P // NS) + j * 128
    pltpu.sync_copy(src0.at[pl.ds(base, 128)], eidx.at[0])
    pltpu.sync_copy(dst0.at[pl.ds(base, 128)], eidx.at[1])
    pltpu.async_copy(table.at[eidx.at[0]], gbuf, sem).wait()
    pltpu.sync_copy(gbuf, acc.at[eidx.at[1]], add=True)
    return _

  lax.fori_loop(0, E0P // NS // 128, body, 0)
  plsc.subcore_barrier()
  pltpu.sync_copy(acc.at[pl.ds(s * 640, 640), :], out.at[pl.ds(s * 640, 640), :])


def _agg0_body(p0, q0, src0, dst0, aggP, aggQ, acc, eidx, gbuf, sem):
  c = lax.axis_index("c")
  s = lax.axis_index("s")

  @pl.when(c == 0)
  def _():
    _agg0_one(p0, aggP, src0, dst0, acc, eidx, gbuf, sem, s)

  @pl.when(c == 1)
  def _():
    _agg0_one(q0, aggQ, src0, dst0, acc, eidx, gbuf, sem, s)


def _run_agg0(p0, q0, src0, dst0):
  out_type = [
      jax.ShapeDtypeStruct((N0P, D), jnp.float32),
      jax.ShapeDtypeStruct((N0P, D), jnp.float32),
  ]
  scratch = [
      pltpu.VMEM_SHARED((N0P, D), jnp.float32),
      pltpu.VMEM((2, 128), jnp.int32),
      pltpu.VMEM((128, D), jnp.float32),
      pltpu.SemaphoreType.DMA,
  ]
  f = pl.kernel(_agg0_body, out_type=out_type, mesh=_mesh(),
                compiler_params=_SC_PARAMS, scratch_types=scratch)
  return f(p0, q0, src0, dst0)


_IOTA = None


def _extract(vec16a, vec16b, r):
  io = lax.iota(jnp.int32, L)
  va = jnp.sum(jnp.where(io == r, vec16a, 0))
  vb = jnp.sum(jnp.where(io + L == r, vec16b, 0))
  return va + vb


def _agg1_one(table, out, src1, dst1, acc, eidx, cbuf, hist, lcur,
              idxg, idxs, gbuf, sem, s):
  io = lax.iota(jnp.int32, L)

  hist[pl.ds(0, L)] = jnp.zeros((L,), jnp.int32)
  hist[pl.ds(L, L)] = jnp.zeros((L,), jnp.int32)

  def h_body(j, _):
    base = s * CE1 + j * 128
    pltpu.sync_copy(dst1.at[pl.ds(base, 128)], eidx.at[1])
    for g in range(8):
      d = eidx[1, pl.ds(g * L, L)]
      bkt = jnp.right_shift(d, 13)
      rank, last = plsc.scan_count(bkt)
      plsc.addupdate_scatter(hist, [bkt], rank, mask=last)
    return _

  lax.fori_loop(0, CE1 // 128, h_body, 0)

  h0v = hist[pl.ds(0, L)]
  h1v = hist[pl.ds(L, L)]
  a0 = jnp.left_shift(jnp.right_shift(h0v + 127, 7), 7)
  a1 = jnp.left_shift(jnp.right_shift(h1v + 127, 7), 7)
  c0 = plsc.cumsum(a0)
  c1v = plsc.cumsum(a1)
  tot0 = jnp.sum(a0)
  lcur[pl.ds(0, L)] = c0 - a0
  lcur[pl.ds(L, L)] = c1v - a1 + tot0
  lstart0 = c0 - a0
  lstart1 = c1v - a1 + tot0

  def p_body(j, _):
    for g in range(8):
      row = (j * 8 + g)
      padS = (jnp.bitwise_and(row * 128 + io * 8, NB - 1)
              + (19 * NB)).astype(jnp.uint32)
      padD = (NB + jnp.bitwise_and(row + io * 8, 127)).astype(jnp.uint32)
      pad = plsc.bitcast(jnp.bitwise_or(padS, jnp.left_shift(padD, 18)),
                         jnp.int32)
      for t in range(8):
        cbuf[row, pl.ds(t * L, L)] = pad
    return _

  lax.fori_loop(0, CROWS // 8, p_body, 0)

  def s_body(j, _):
    base = s * CE1 + j * 128
    pltpu.sync_copy(src1.at[pl.ds(base, 128)], eidx.at[0])
    pltpu.sync_copy(dst1.at[pl.ds(base, 128)], eidx.at[1])
    for g in range(8):
      sv = eidx[0, pl.ds(g * L, L)]
      d = eidx[1, pl.ds(g * L, L)]
      bkt = jnp.right_shift(d, 13)
      drel = jnp.bitwise_and(d, NB - 1)
      packed = plsc.bitcast(
          jnp.bitwise_or(sv.astype(jnp.uint32),
                         jnp.left_shift(drel.astype(jnp.uint32), 18)),
          jnp.int32)
      rank, last = plsc.scan_count(bkt)
      basep = plsc.load_gather(lcur, [bkt])
      pos = basep + rank - 1
      hi = jnp.right_shift(pos, 7)
      lo = jnp.bitwise_and(pos, 127)
      plsc.store_scatter(cbuf, [hi, lo], packed)
      plsc.store_scatter(lcur, [bkt], pos + 1, mask=last)
    return _

  lax.fori_loop(0, CE1 // 128, s_body, 0)

  def r_body(r, _):
    pltpu.sync_copy(table.at[pl.ds(r * NB + s * 512, 512), :],
                    acc.at[pl.ds(s * 512, 512), :])
    plsc.subcore_barrier()
    cnt = _extract(h0v, h1v, r)
    start = _extract(lstart0, lstart1, r)
    row0 = jnp.right_shift(start, 7)
    nseg = jnp.right_shift(cnt + 127, 7)

    def seg_body(j, _2):
      row = row0 + j
      for t in range(8):
        packed = plsc.bitcast(cbuf[row, pl.ds(t * L, L)], jnp.uint32)
        idxg[0, pl.ds(t * L, L)] = jnp.bitwise_and(
            packed, jnp.uint32(0x3FFFF)).astype(jnp.int32)
        idxs[0, pl.ds(t * L, L)] = jnp.right_shift(packed, 18).astype(jnp.int32)
      pltpu.async_copy(table.at[idxg.at[0]], gbuf, sem).wait()
      pltpu.sync_copy(gbuf, acc.at[idxs.at[0]], add=True)
      return _2

    lax.fori_loop(0, nseg, seg_body, 0)
    plsc.subcore_barrier()
    pltpu.sync_copy(acc.at[pl.ds(s * 512, 512), :],
                    out.at[pl.ds(r * NB + s * 512, 512), :])
    plsc.subcore_barrier()
    return _

  lax.fori_loop(0, NBK, r_body, 0)


def _agg1_body(p1, q1, src1, dst1, aggP, aggQ, acc, eidx, cbuf,
               hist, lcur, idxg, idxs, gbuf, sem):
  c = lax.axis_index("c")
  s = lax.axis_index("s")

  @pl.when(c == 0)
  def _():
    _agg1_one(p1, aggP, src1, dst1, acc, eidx, cbuf, hist, lcur,
              idxg, idxs, gbuf, sem, s)

  @pl.when(c == 1)
  def _():
    _agg1_one(q1, aggQ, src1, dst1, acc, eidx, cbuf, hist, lcur,
              idxg, idxs, gbuf, sem, s)


def _run_agg1(p1, q1, src1, dst1):
  out_type = [
      jax.ShapeDtypeStruct((N1P, D), jnp.float32),
      jax.ShapeDtypeStruct((N1P, D), jnp.float32),
  ]
  scratch = [
      pltpu.VMEM_SHARED((NB + 128, D), jnp.float32),
      pltpu.VMEM((2, 128), jnp.int32),
      pltpu.VMEM((CROWS, 128), jnp.int32),
      pltpu.VMEM((2 * L,), jnp.int32),
      pltpu.VMEM((2 * L,), jnp.int32),
      pltpu.VMEM((1, 128), jnp.int32),
      pltpu.VMEM((1, 128), jnp.int32),
      pltpu.VMEM((128, D), jnp.float32),
      pltpu.SemaphoreType.DMA,
  ]
  f = pl.kernel(_agg1_body, out_type=out_type, mesh=_mesh(),
                compiler_params=_SC_PARAMS, scratch_types=scratch)
  return f(p1, q1, src1, dst1)


def _b_level0_kern(h_ref, td0_ref, td1_ref, hA0_ref, hA1_ref, p_ref, q_ref):
  ns = lax.rsqrt(hA0_ref[...] + hA1_ref[...] + 1.0)
  p_ref[...] = h_ref[...] * ns
  q_ref[...] = (td0_ref[...] + td1_ref[...]) * ns


def _b_level1_kern(h_ref, bu_ref, hC0_ref, hC1_ref, p_ref, q_ref):
  ns = lax.rsqrt(hC0_ref[...] + hC1_ref[...] + 1.0)
  p_ref[...] = h_ref[...] * ns
  q_ref[...] = bu_ref[...] * ns


def _run_b(kern, h, fus_args, hh0, hh1, n):
  nb = n // BLK
  row = pl.BlockSpec((BLK, D), lambda i: (i, 0))
  col = pl.BlockSpec((BLK, 1), lambda i: (i, 0))
  in_specs = [row] + [row] * len(fus_args) + [col, col]
  return pl.pallas_call(
      kern,
      grid=(nb,),
      in_specs=in_specs,
      out_specs=[row, row],
      out_shape=[jax.ShapeDtypeStruct((n, D), jnp.float32),
                 jax.ShapeDtypeStruct((n, D), jnp.float32)],
  )(h, *fus_args, hh0, hh1)


def _c_kern(aggP_ref, aggQ_ref, hd0_ref, hd1_ref, Wc_ref, bc_ref, Wf_ref,
            bf_ref, Wr_ref, Wl_ref, bcat_ref, g_ref, b_ref, out_ref):
  nd = lax.rsqrt(hd0_ref[...] + hd1_ref[...] + 1.0)
  cs = jnp.dot(aggP_ref[...] * nd, Wc_ref[...],
               preferred_element_type=jnp.float32) + bc_ref[...]
  fs = jnp.dot(aggQ_ref[...] * nd, Wf_ref[...],
               preferred_element_type=jnp.float32) + bf_ref[...]
  r1 = jnp.maximum(cs, 0.0) + jnp.maximum(fs, 0.0)
  r2 = cs + fs
  res = (jnp.dot(r1, Wr_ref[...], preferred_element_type=jnp.float32)
         + jnp.dot(r2, Wl_ref[...], preferred_element_type=jnp.float32)
         + bcat_ref[...])
  mu = jnp.mean(res, axis=-1, keepdims=True)
  var = jnp.mean(jnp.square(res - mu), axis=-1, keepdims=True)
  out_ref[...] = (res - mu) * lax.rsqrt(var + 1e-5) * g_ref[...] + b_ref[...]


def _run_c(aggP, aggQ, hd0, hd1, Wc, bc, Wf, bf, Wr, Wl, bcat, g, b, n):
  nb = n // BLK
  row = pl.BlockSpec((BLK, D), lambda i: (i, 0))
  col = pl.BlockSpec((BLK, 1), lambda i: (i, 0))
  wmat = pl.BlockSpec((D, D), lambda i: (0, 0))
  wrow = pl.BlockSpec((1, D), lambda i: (0, 0))
  return pl.pallas_call(
      _c_kern,
      grid=(nb,),
      in_specs=[row, row, col, col, wmat, wrow, wmat, wrow, wmat, wmat,
                wrow, wrow, wrow],
      out_specs=row,
      out_shape=jax.ShapeDtypeStruct((n, D), jnp.float32),
  )(aggP, aggQ, hd0, hd1, Wc, bc, Wf, bf, Wr, Wl, bcat, g, b)


def _pad_edges(src, dst, e, ep, n, npad):
  npad_lo = n
  span = npad - n
  i = jnp.arange(ep - e, dtype=jnp.int32)
  fill = npad_lo + (i % span)
  srcp = jnp.concatenate([src, fill])
  dstp = jnp.concatenate([dst, fill])
  return srcp, dstp


def kernel(h0, h1, edge_index0, edge_index1,
           W_conv_td, b_conv_td, W_fus_td, b_fus_td, conv_w_td, td_w,
           Wcat_td, bcat_td, ln_g_td, ln_b_td,
           W_conv_bu, b_conv_bu, W_fus_bu, b_fus_bu, conv_w_bu, bu_w,
           Wcat_bu, bcat_bu, ln_g_bu, ln_b_bu):
  f32 = jnp.float32
  h0p = jnp.zeros((N0P, D), f32).at[:N0].set(h0)
  h1p = jnp.zeros((N1P, D), f32).at[:N1].set(h1)
  src0, dst0 = edge_index0[0], edge_index0[1]
  src1, dst1 = edge_index1[0], edge_index1[1]
  src0p, dst0p = _pad_edges(src0, dst0, E0, E0P, N0, N0P)
  src1p, dst1p = _pad_edges(src1, dst1, E1, E1P, N1, N1P)

  histA, histB, histC, histD, tdp, bu = _run_pass_a(
      src0p, dst0p, src1p, dst1p, h0p, h1p)

  hA0 = histA[0][:, None]
  hA1 = histA[1][:, None]
  hB0 = histB[0][:, None]
  hB1 = histB[1][:, None]
  hC0 = histC[0][:, None]
  hC1 = histC[1][:, None]
  hD0 = histD[0][:, None]
  hD1 = histD[1][:, None]

  p0, q0 = _run_b(_b_level0_kern, h0p, [tdp[0], tdp[1]], hA0, hA1, N0P)
  p1, q1 = _run_b(_b_level1_kern, h1p, [bu], hC0, hC1, N1P)

  aggP0, aggQ0 = _run_agg0(p0, q0, src0p, dst0p)
  aggP1, aggQ1 = _run_agg1(p1, q1, src1p, dst1p)

  Wc_td = W_conv_td * conv_w_td[None, :]
  bc_td = (b_conv_td * conv_w_td)[None, :]
  Wf_td = W_fus_td * td_w[None, :]
  bf_td = (b_fus_td * td_w)[None, :]
  Wc_bu = W_conv_bu * conv_w_bu[None, :]
  bc_bu = (b_conv_bu * conv_w_bu)[None, :]
  Wf_bu = W_fus_bu * bu_w[None, :]
  bf_bu = (b_fus_bu * bu_w)[None, :]

  new_h0 = _run_c(aggP0, aggQ0, hB0, hB1, Wc_td, bc_td, Wf_td, bf_td,
                  Wcat_td[:D], Wcat_td[D:], bcat_td[None, :],
                  ln_g_td[None, :], ln_b_td[None, :], N0P)
  new_h1 = _run_c(aggP1, aggQ1, hD0, hD1, Wc_bu, bc_bu, Wf_bu, bf_bu,
                  Wcat_bu[:D], Wcat_bu[D:], bcat_bu[None, :],
                  ln_g_bu[None, :], ln_b_bu[None, :], N1P)

  return (new_h0[:N0], new_h1[:N1])


# speedup vs baseline: 8.8676x; 1.2164x over previous
import functools

import jax
import jax.numpy as jnp
from jax import lax
from jax.experimental import pallas as pl
from jax.experimental.pallas import tpu as pltpu
from jax.experimental.pallas import tpu_sc as plsc

N0, E0 = 10000, 160000
N1, E1 = 160000, 480000
D = 128

NC, NS, L = 2, 16, 16
N0P = 10240
N1P = 163840
E0P = 163840
E1P = 491520
C0 = E0P // (NC * NS)
C1 = E1P // (NC * NS)
NB = 8192
NBK = N1P // NB
CE1 = E1P // NS
CROWS = 264
BLK = 1024


_SC_PARAMS = pltpu.CompilerParams(needs_layout_passes=False)


def _mesh():
  return plsc.VectorSubcoreMesh(core_axis_name="c", subcore_axis_name="s",
                                num_cores=NC, num_subcores=NS)


def _pass_a0_body(src0, dst0, h0p, h1p, zflat, zrows, ones,
                  histA, histB, tdp, bu,
                  hA, hB, td,
                  eidx, onev, rbuf, gbuf, sem):
  c = lax.axis_index("c")
  s = lax.axis_index("s")
  wid = c * NS + s

  pltpu.sync_copy(zflat.at[pl.ds(0, 640)], hA.at[pl.ds(s * 640, 640)])
  pltpu.sync_copy(zflat.at[pl.ds(0, 640)], hB.at[pl.ds(s * 640, 640)])
  pltpu.sync_copy(zrows, td.at[pl.ds(s * 640, 640), :])
  pltpu.sync_copy(ones, onev.at[0])
  plsc.subcore_barrier()

  def body0(j, _):
    base = wid * C0 + j * 128
    pltpu.sync_copy(src0.at[pl.ds(base, 128)], eidx.at[0])
    pltpu.sync_copy(dst0.at[pl.ds(base, 128)], eidx.at[1])
    pltpu.sync_copy(onev.at[0], hA.at[eidx.at[0]], add=True)
    pltpu.sync_copy(onev.at[0], hB.at[eidx.at[1]], add=True)
    pltpu.sync_copy(h1p.at[pl.ds(base, 128), :], rbuf)
    pltpu.sync_copy(rbuf, td.at[eidx.at[0]], add=True)
    pltpu.async_copy(h0p.at[eidx.at[0]], gbuf, sem).wait()
    pltpu.sync_copy(gbuf, bu.at[pl.ds(base, 128), :])
    return _

  lax.fori_loop(0, C0 // 128, body0, 0)
  plsc.subcore_barrier()

  pltpu.sync_copy(hA.at[pl.ds(s * 640, 640)], histA.at[c, pl.ds(s * 640, 640)])
  pltpu.sync_copy(hB.at[pl.ds(s * 640, 640)], histB.at[c, pl.ds(s * 640, 640)])
  pltpu.sync_copy(td.at[pl.ds(s * 640, 640), :],
                  tdp.at[c, pl.ds(s * 640, 640), :])


def _pass_a1_body(src1r, dst1r, zflat, ones,
                  histC, histD,
                  hC, hD,
                  sbufA, sbufB, onev):
  c = lax.axis_index("c")
  s = lax.axis_index("s")
  wid = c * NS + s

  pltpu.sync_copy(zflat, hC.at[pl.ds(s * 10240, 8192)])
  pltpu.sync_copy(zflat.at[pl.ds(0, 2048)],
                  hC.at[pl.ds(s * 10240 + 8192, 2048)])
  pltpu.sync_copy(zflat, hD.at[pl.ds(s * 10240, 8192)])
  pltpu.sync_copy(zflat.at[pl.ds(0, 2048)],
                  hD.at[pl.ds(s * 10240 + 8192, 2048)])
  pltpu.sync_copy(ones, onev.at[0])
  pltpu.sync_copy(src1r.at[pl.ds(wid * 120, 120), :], sbufA)
  pltpu.sync_copy(dst1r.at[pl.ds(wid * 120, 120), :], sbufB)
  plsc.subcore_barrier()

  def body1(j, _):
    pltpu.sync_copy(onev.at[0], hC.at[sbufA.at[j]], add=True)
    pltpu.sync_copy(onev.at[0], hD.at[sbufB.at[j]], add=True)
    return _

  lax.fori_loop(0, 120, body1, 0)
  plsc.subcore_barrier()

  pltpu.sync_copy(hC.at[pl.ds(s * 10240, 10240)],
                  histC.at[c, pl.ds(s * 10240, 10240)])
  pltpu.sync_copy(hD.at[pl.ds(s * 10240, 10240)],
                  histD.at[c, pl.ds(s * 10240, 10240)])


def _run_pass_a(src0, dst0, src1, dst1, h0p, h1p):
  zflat = jnp.zeros((8192,), jnp.float32)
  zrows = jnp.zeros((640, D), jnp.float32)
  ones = jnp.ones((128,), jnp.float32)
  out0 = [
      jax.ShapeDtypeStruct((NC, N0P), jnp.float32),
      jax.ShapeDtypeStruct((NC, N0P), jnp.float32),
      jax.ShapeDtypeStruct((NC, N0P, D), jnp.float32),
      jax.ShapeDtypeStruct((N1P, D), jnp.float32),
  ]
  scratch0 = [
      pltpu.VMEM_SHARED((N0P,), jnp.float32),
      pltpu.VMEM_SHARED((N0P,), jnp.float32),
      pltpu.VMEM_SHARED((N0P, D), jnp.float32),
      pltpu.VMEM((2, 128), jnp.int32),
      pltpu.VMEM((1, 128), jnp.float32),
      pltpu.VMEM((128, D), jnp.float32),
      pltpu.VMEM((128, D), jnp.float32),
      pltpu.SemaphoreType.DMA,
  ]
  f0 = pl.kernel(_pass_a0_body, out_type=out0, mesh=_mesh(),
                 compiler_params=_SC_PARAMS, scratch_types=scratch0)
  histA, histB, tdp, bu = f0(src0, dst0, h0p, h1p, zflat, zrows, ones)

  out1 = [
      jax.ShapeDtypeStruct((NC, N1P), jnp.float32),
      jax.ShapeDtypeStruct((NC, N1P), jnp.float32),
  ]
  scratch1 = [
      pltpu.VMEM_SHARED((N1P,), jnp.float32),
      pltpu.VMEM_SHARED((N1P,), jnp.float32),
      pltpu.VMEM((120, 128), jnp.int32),
      pltpu.VMEM((120, 128), jnp.int32),
      pltpu.VMEM((1, 128), jnp.float32),
  ]
  f1 = pl.kernel(_pass_a1_body, out_type=out1, mesh=_mesh(),
                 compiler_params=_SC_PARAMS, scratch_types=scratch1)
  histC, histD = f1(src1.reshape(E1P // 128, 128), dst1.reshape(E1P // 128, 128),
                    zflat, ones)
  return histA, histB, histC, histD, tdp, bu


def _agg0_one(table, out, src0, dst0, acc, eidx, gbuf, sem, s):
  pltpu.sync_copy(table.at[pl.ds(s * 640, 640), :], acc.at[pl.ds(s * 640, 640), :])
  plsc.subcore_barrier()

  def body(j, _):
    base = s * (E0P // NS) + j * 128
    pltpu.sync_copy(src0.at[pl.ds(base, 128)], eidx.at[0])
    pltpu.sync_copy(dst0.at[pl.ds(base, 128)], eidx.at[1])
    pltpu.async_copy(table.at[eidx.at[0]], gbuf, sem).wait()
    pltpu.sync_copy(gbuf, acc.at[eidx.at[1]], add=True)
    return _

  lax.fori_loop(0, E0P // NS // 128, body, 0)
  plsc.subcore_barrier()
  pltpu.sync_copy(acc.at[pl.ds(s * 640, 640), :], out.at[pl.ds(s * 640, 640), :])


def _agg0_body(p0, q0, src0, dst0, aggP, aggQ, acc, eidx, gbuf, sem):
  c = lax.axis_index("c")
  s = lax.axis_index("s")

  @pl.when(c == 0)
  def _():
    _agg0_one(p0, aggP, src0, dst0, acc, eidx, gbuf, sem, s)

  @pl.when(c == 1)
  def _():
    _agg0_one(q0, aggQ, src0, dst0, acc, eidx, gbuf, sem, s)


def _run_agg0(p0, q0, src0, dst0):
  out_type = [
      jax.ShapeDtypeStruct((N0P, D), jnp.float32),
      jax.ShapeDtypeStruct((N0P, D), jnp.float32),
  ]
  scratch = [
      pltpu.VMEM_SHARED((N0P, D), jnp.float32),
      pltpu.VMEM((2, 128), jnp.int32),
      pltpu.VMEM((128, D), jnp.float32),
      pltpu.SemaphoreType.DMA,
  ]
  f = pl.kernel(_agg0_body, out_type=out_type, mesh=_mesh(),
                compiler_params=_SC_PARAMS, scratch_types=scratch)
  return f(p0, q0, src0, dst0)


_IOTA = None


def _extract(vec16a, vec16b, r):
  io = lax.iota(jnp.int32, L)
  va = jnp.sum(jnp.where(io == r, vec16a, 0))
  vb = jnp.sum(jnp.where(io + L == r, vec16b, 0))
  return va + vb


def _agg1_one(table, out, src1r, dst1r, acc, sbuf, dbuf, cbuf, hist, lcur,
              idxgA, idxgB, idxsA, idxsB, gbuf, semA, semS, s):
  io = lax.iota(jnp.int32, L)

  hist[pl.ds(0, L)] = jnp.zeros((L,), jnp.int32)
  hist[pl.ds(L, L)] = jnp.zeros((L,), jnp.int32)

  def h_outer(jb, _):
    pltpu.sync_copy(dst1r.at[pl.ds(s * 240 + jb * 24, 24), :], dbuf)

    def h_in(j, _2):
      for g in range(8):
        d = dbuf[j, pl.ds(g * L, L)]
        bkt = jnp.right_shift(d, 13)
        rank, last = plsc.scan_count(bkt)
        plsc.addupdate_scatter(hist, [bkt], rank, mask=last)
      return _2

    lax.fori_loop(0, 24, h_in, 0)
    return _

  lax.fori_loop(0, 10, h_outer, 0)

  h0v = hist[pl.ds(0, L)]
  h1v = hist[pl.ds(L, L)]
  a0 = jnp.left_shift(jnp.right_shift(h0v + 63, 6), 6)
  a1 = jnp.left_shift(jnp.right_shift(h1v + 63, 6), 6)
  c0 = plsc.cumsum(a0)
  c1v = plsc.cumsum(a1)
  tot0 = jnp.sum(a0)
  lcur[pl.ds(0, L)] = c0 - a0
  lcur[pl.ds(L, L)] = c1v - a1 + tot0
  lstart0 = c0 - a0
  lstart1 = c1v - a1 + tot0

  def p_body(j, _):
    for g in range(8):
      row = (j * 8 + g)
      padS = (jnp.bitwise_and(row * 128 + io * 8, NB - 1)
              + (19 * NB)).astype(jnp.uint32)
      padD = (NB + jnp.bitwise_and(row + io * 8, 127)).astype(jnp.uint32)
      pad = plsc.bitcast(jnp.bitwise_or(padS, jnp.left_shift(padD, 18)),
                         jnp.int32)
      for t in range(8):
        cbuf[row, pl.ds(t * L, L)] = pad
    return _

  lax.fori_loop(0, CROWS // 8, p_body, 0)

  def s_outer(jb, _):
    pltpu.sync_copy(src1r.at[pl.ds(s * 240 + jb * 24, 24), :], sbuf)
    pltpu.sync_copy(dst1r.at[pl.ds(s * 240 + jb * 24, 24), :], dbuf)

    def s_in(j, _2):
      for g in range(8):
        sv = sbuf[j, pl.ds(g * L, L)]
        d = dbuf[j, pl.ds(g * L, L)]
        bkt = jnp.right_shift(d, 13)
        drel = jnp.bitwise_and(d, NB - 1)
        packed = plsc.bitcast(
            jnp.bitwise_or(sv.astype(jnp.uint32),
                           jnp.left_shift(drel.astype(jnp.uint32), 18)),
            jnp.int32)
        rank, last = plsc.scan_count(bkt)
        basep = plsc.load_gather(lcur, [bkt])
        pos = basep + rank - 1
        hi = jnp.right_shift(pos, 7)
        lo = jnp.bitwise_and(pos, 127)
        plsc.store_scatter(cbuf, [hi, lo], packed)
        plsc.store_scatter(lcur, [bkt], pos + 1, mask=last)
      return _2

    lax.fori_loop(0, 24, s_in, 0)
    return _

  lax.fori_loop(0, 10, s_outer, 0)

  def _drain_one():
    pltpu.make_async_copy(table.at[pl.ds(0, 64), :], gbuf.at[0], semS).wait()

  def r_body(r, _):
    pltpu.sync_copy(table.at[pl.ds(r * NB + s * 512, 512), :],
                    acc.at[pl.ds(s * 512, 512), :])
    plsc.subcore_barrier()
    cnt = _extract(h0v, h1v, r)
    start = _extract(lstart0, lstart1, r)
    blk0 = jnp.right_shift(start, 6)
    nseg = jnp.right_shift(cnt + 63, 6)

    def seg_body(j, _2):
      slot = jnp.bitwise_and(j, 1)
      ab = blk0 + j
      row = jnp.right_shift(ab, 1)
      parity = jnp.bitwise_and(ab, 1)

      @pl.when(j >= 2)
      def _w():
        _drain_one()

      def _unpack(off, idxg_r, idxs_r):
        for t in range(4):
          packed = plsc.bitcast(cbuf[row, pl.ds(off + t * L, L)], jnp.uint32)
          idxg_r[0, pl.ds(t * L, L)] = jnp.bitwise_and(
              packed, jnp.uint32(0x3FFFF)).astype(jnp.int32)
          idxs_r[0, pl.ds(t * L, L)] = jnp.right_shift(
              packed, 18).astype(jnp.int32)

      def _do(idxg_r, idxs_r, gslot):
        @pl.when(parity == 0)
        def _u0():
          _unpack(0, idxg_r, idxs_r)

        @pl.when(parity == 1)
        def _u1():
          _unpack(64, idxg_r, idxs_r)

        pltpu.async_copy(table.at[idxg_r.at[0]], gbuf.at[gslot], semA).wait()
        pltpu.async_copy(gbuf.at[gslot], acc.at[idxs_r.at[0]], semS, add=True)

      @pl.when(slot == 0)
      def _s0():
        _do(idxgA, idxsA, 0)

      @pl.when(slot == 1)
      def _s1():
        _do(idxgB, idxsB, 1)

      return _2

    lax.fori_loop(0, nseg, seg_body, 0)

    def d_body(j, _2):
      _drain_one()
      return _2

    lax.fori_loop(0, jnp.minimum(nseg, 2), d_body, 0)
    plsc.subcore_barrier()
    pltpu.sync_copy(acc.at[pl.ds(s * 512, 512), :],
                    out.at[pl.ds(r * NB + s * 512, 512), :])
    plsc.subcore_barrier()
    return _

  lax.fori_loop(0, NBK, r_body, 0)


def _agg1_body(p1, q1, src1r, dst1r, aggP, aggQ, acc, sbuf, dbuf, cbuf,
               hist, lcur, idxgA, idxgB, idxsA, idxsB, gbuf, semA, semS):
  c = lax.axis_index("c")
  s = lax.axis_index("s")

  @pl.when(c == 0)
  def _():
    _agg1_one(p1, aggP, src1r, dst1r, acc, sbuf, dbuf, cbuf, hist, lcur,
              idxgA, idxgB, idxsA, idxsB, gbuf, semA, semS, s)

  @pl.when(c == 1)
  def _():
    _agg1_one(q1, aggQ, src1r, dst1r, acc, sbuf, dbuf, cbuf, hist, lcur,
              idxgA, idxgB, idxsA, idxsB, gbuf, semA, semS, s)


def _run_agg1(p1, q1, src1r, dst1r):
  out_type = [
      jax.ShapeDtypeStruct((N1P, D), jnp.float32),
      jax.ShapeDtypeStruct((N1P, D), jnp.float32),
  ]
  scratch = [
      pltpu.VMEM_SHARED((NB + 128, D), jnp.float32),
      pltpu.VMEM((24, 128), jnp.int32),
      pltpu.VMEM((24, 128), jnp.int32),
      pltpu.VMEM((CROWS, 128), jnp.int32),
      pltpu.VMEM((2 * L,), jnp.int32),
      pltpu.VMEM((2 * L,), jnp.int32),
      pltpu.VMEM((1, 64), jnp.int32),
      pltpu.VMEM((1, 64), jnp.int32),
      pltpu.VMEM((1, 64), jnp.int32),
      pltpu.VMEM((1, 64), jnp.int32),
      pltpu.VMEM((2, 64, D), jnp.float32),
      pltpu.SemaphoreType.DMA,
      pltpu.SemaphoreType.DMA,
  ]
  f = pl.kernel(_agg1_body, out_type=out_type, mesh=_mesh(),
                compiler_params=_SC_PARAMS, scratch_types=scratch)
  return f(p1, q1, src1r, dst1r)


def _b_level0_kern(h_ref, td0_ref, td1_ref, hA0_ref, hA1_ref, p_ref, q_ref):
  ns = lax.rsqrt(hA0_ref[...] + hA1_ref[...] + 1.0)
  p_ref[...] = h_ref[...] * ns
  q_ref[...] = (td0_ref[...] + td1_ref[...]) * ns


def _b_level1_kern(h_ref, bu_ref, hC0_ref, hC1_ref, p_ref, q_ref):
  ns = lax.rsqrt(hC0_ref[...] + hC1_ref[...] + 1.0)
  p_ref[...] = h_ref[...] * ns
  q_ref[...] = bu_ref[...] * ns


def _run_b(kern, h, fus_args, hh0, hh1, n):
  nb = n // BLK
  row = pl.BlockSpec((BLK, D), lambda i: (i, 0))
  col = pl.BlockSpec((BLK, 1), lambda i: (i, 0))
  in_specs = [row] + [row] * len(fus_args) + [col, col]
  return pl.pallas_call(
      kern,
      grid=(nb,),
      in_specs=in_specs,
      out_specs=[row, row],
      out_shape=[jax.ShapeDtypeStruct((n, D), jnp.float32),
                 jax.ShapeDtypeStruct((n, D), jnp.float32)],
  )(h, *fus_args, hh0, hh1)


def _c_kern(aggP_ref, aggQ_ref, hd0_ref, hd1_ref, Wc_ref, bc_ref, Wf_ref,
            bf_ref, Wr_ref, Wl_ref, bcat_ref, g_ref, b_ref, out_ref):
  nd = lax.rsqrt(hd0_ref[...] + hd1_ref[...] + 1.0)
  cs = jnp.dot(aggP_ref[...] * nd, Wc_ref[...],
               preferred_element_type=jnp.float32) + bc_ref[...]
  fs = jnp.dot(aggQ_ref[...] * nd, Wf_ref[...],
               preferred_element_type=jnp.float32) + bf_ref[...]
  r1 = jnp.maximum(cs, 0.0) + jnp.maximum(fs, 0.0)
  r2 = cs + fs
  res = (jnp.dot(r1, Wr_ref[...], preferred_element_type=jnp.float32)
         + jnp.dot(r2, Wl_ref[...], preferred_element_type=jnp.float32)
         + bcat_ref[...])
  mu = jnp.mean(res, axis=-1, keepdims=True)
  var = jnp.mean(jnp.square(res - mu), axis=-1, keepdims=True)
  out_ref[...] = (res - mu) * lax.rsqrt(var + 1e-5) * g_ref[...] + b_ref[...]


def _run_c(aggP, aggQ, hd0, hd1, Wc, bc, Wf, bf, Wr, Wl, bcat, g, b, n):
  nb = n // BLK
  row = pl.BlockSpec((BLK, D), lambda i: (i, 0))
  col = pl.BlockSpec((BLK, 1), lambda i: (i, 0))
  wmat = pl.BlockSpec((D, D), lambda i: (0, 0))
  wrow = pl.BlockSpec((1, D), lambda i: (0, 0))
  return pl.pallas_call(
      _c_kern,
      grid=(nb,),
      in_specs=[row, row, col, col, wmat, wrow, wmat, wrow, wmat, wmat,
                wrow, wrow, wrow],
      out_specs=row,
      out_shape=jax.ShapeDtypeStruct((n, D), jnp.float32),
  )(aggP, aggQ, hd0, hd1, Wc, bc, Wf, bf, Wr, Wl, bcat, g, b)


def _pad_edges(src, dst, e, ep, n, npad):
  npad_lo = n
  span = npad - n
  i = jnp.arange(ep - e, dtype=jnp.int32)
  fill = npad_lo + (i % span)
  srcp = jnp.concatenate([src, fill])
  dstp = jnp.concatenate([dst, fill])
  return srcp, dstp


def kernel(h0, h1, edge_index0, edge_index1,
           W_conv_td, b_conv_td, W_fus_td, b_fus_td, conv_w_td, td_w,
           Wcat_td, bcat_td, ln_g_td, ln_b_td,
           W_conv_bu, b_conv_bu, W_fus_bu, b_fus_bu, conv_w_bu, bu_w,
           Wcat_bu, bcat_bu, ln_g_bu, ln_b_bu):
  f32 = jnp.float32
  h0p = jnp.zeros((N0P, D), f32).at[:N0].set(h0)
  h1p = jnp.zeros((N1P, D), f32).at[:N1].set(h1)
  src0, dst0 = edge_index0[0], edge_index0[1]
  src1, dst1 = edge_index1[0], edge_index1[1]
  src0p, dst0p = _pad_edges(src0, dst0, E0, E0P, N0, N0P)
  src1p, dst1p = _pad_edges(src1, dst1, E1, E1P, N1, N1P)

  histA, histB, histC, histD, tdp, bu = _run_pass_a(
      src0p, dst0p, src1p, dst1p, h0p, h1p)

  hA0 = histA[0][:, None]
  hA1 = histA[1][:, None]
  hB0 = histB[0][:, None]
  hB1 = histB[1][:, None]
  hC0 = histC[0][:, None]
  hC1 = histC[1][:, None]
  hD0 = histD[0][:, None]
  hD1 = histD[1][:, None]

  p0, q0 = _run_b(_b_level0_kern, h0p, [tdp[0], tdp[1]], hA0, hA1, N0P)
  p1, q1 = _run_b(_b_level1_kern, h1p, [bu], hC0, hC1, N1P)

  aggP0, aggQ0 = _run_agg0(p0, q0, src0p, dst0p)
  aggP1, aggQ1 = _run_agg1(p1, q1, src1p.reshape(E1P // 128, 128),
                           dst1p.reshape(E1P // 128, 128))

  Wc_td = W_conv_td * conv_w_td[None, :]
  bc_td = (b_conv_td * conv_w_td)[None, :]
  Wf_td = W_fus_td * td_w[None, :]
  bf_td = (b_fus_td * td_w)[None, :]
  Wc_bu = W_conv_bu * conv_w_bu[None, :]
  bc_bu = (b_conv_bu * conv_w_bu)[None, :]
  Wf_bu = W_fus_bu * bu_w[None, :]
  bf_bu = (b_fus_bu * bu_w)[None, :]

  new_h0 = _run_c(aggP0, aggQ0, hB0, hB1, Wc_td, bc_td, Wf_td, bf_td,
                  Wcat_td[:D], Wcat_td[D:], bcat_td[None, :],
                  ln_g_td[None, :], ln_b_td[None, :], N0P)
  new_h1 = _run_c(aggP1, aggQ1, hD0, hD1, Wc_bu, bc_bu, Wf_bu, bf_bu,
                  Wcat_bu[:D], Wcat_bu[D:], bcat_bu[None, :],
                  ln_g_bu[None, :], ln_b_bu[None, :], N1P)

  return (new_h0[:N0], new_h1[:N1])


# speedup vs baseline: 9.1372x; 1.0304x over previous
import functools

import jax
import jax.numpy as jnp
from jax import lax
from jax.experimental import pallas as pl
from jax.experimental.pallas import tpu as pltpu
from jax.experimental.pallas import tpu_sc as plsc

N0, E0 = 10000, 160000
N1, E1 = 160000, 480000
D = 128

NC, NS, L = 2, 16, 16
N0P = 10240
N1P = 163840
E0P = 163840
E1P = 491520
C0 = E0P // (NC * NS)
C1 = E1P // (NC * NS)
NB = 8192
NBK = N1P // NB
CE1 = E1P // NS
CROWS = 264
BLK = 1024


_SC_PARAMS = pltpu.CompilerParams(needs_layout_passes=False)


def _mesh():
  return plsc.VectorSubcoreMesh(core_axis_name="c", subcore_axis_name="s",
                                num_cores=NC, num_subcores=NS)


def _pass_a0_body(src0r, dst0r, h0p, h1p, zflat, zrows, ones,
                  histA, histB, tdp, bu,
                  hA, hB, td,
                  sbuf, dbuf, onev, rbuf, gbuf, sem):
  c = lax.axis_index("c")
  s = lax.axis_index("s")
  wid = c * NS + s

  pltpu.sync_copy(zflat.at[pl.ds(0, 640)], hA.at[pl.ds(s * 640, 640)])
  pltpu.sync_copy(zflat.at[pl.ds(0, 640)], hB.at[pl.ds(s * 640, 640)])
  pltpu.sync_copy(zrows, td.at[pl.ds(s * 640, 640), :])
  pltpu.sync_copy(ones, onev.at[0])
  pltpu.sync_copy(src0r.at[pl.ds(wid * 40, 40), :], sbuf)
  pltpu.sync_copy(dst0r.at[pl.ds(wid * 40, 40), :], dbuf)
  plsc.subcore_barrier()

  def body0(j, _):
    base = wid * C0 + j * 128
    pltpu.sync_copy(onev.at[0], hA.at[sbuf.at[j]], add=True)
    pltpu.sync_copy(onev.at[0], hB.at[dbuf.at[j]], add=True)
    pltpu.sync_copy(h1p.at[pl.ds(base, 128), :], rbuf)
    pltpu.sync_copy(rbuf, td.at[sbuf.at[j]], add=True)
    pltpu.async_copy(h0p.at[sbuf.at[j]], gbuf, sem).wait()
    pltpu.sync_copy(gbuf, bu.at[pl.ds(base, 128), :])
    return _

  lax.fori_loop(0, C0 // 128, body0, 0)
  plsc.subcore_barrier()

  pltpu.sync_copy(hA.at[pl.ds(s * 640, 640)], histA.at[c, pl.ds(s * 640, 640)])
  pltpu.sync_copy(hB.at[pl.ds(s * 640, 640)], histB.at[c, pl.ds(s * 640, 640)])
  pltpu.sync_copy(td.at[pl.ds(s * 640, 640), :],
                  tdp.at[c, pl.ds(s * 640, 640), :])


def _pass_a1_body(src1r, dst1r, zflat, ones,
                  histC, histD,
                  hC, hD,
                  sbufA, sbufB, onev):
  c = lax.axis_index("c")
  s = lax.axis_index("s")
  wid = c * NS + s

  pltpu.sync_copy(zflat, hC.at[pl.ds(s * 10240, 8192)])
  pltpu.sync_copy(zflat.at[pl.ds(0, 2048)],
                  hC.at[pl.ds(s * 10240 + 8192, 2048)])
  pltpu.sync_copy(zflat, hD.at[pl.ds(s * 10240, 8192)])
  pltpu.sync_copy(zflat.at[pl.ds(0, 2048)],
                  hD.at[pl.ds(s * 10240 + 8192, 2048)])
  pltpu.sync_copy(ones, onev.at[0])
  pltpu.sync_copy(src1r.at[pl.ds(wid * 120, 120), :], sbufA)
  pltpu.sync_copy(dst1r.at[pl.ds(wid * 120, 120), :], sbufB)
  plsc.subcore_barrier()

  def body1(j, _):
    pltpu.sync_copy(onev.at[0], hC.at[sbufA.at[j]], add=True)
    pltpu.sync_copy(onev.at[0], hD.at[sbufB.at[j]], add=True)
    return _

  lax.fori_loop(0, 120, body1, 0)
  plsc.subcore_barrier()

  pltpu.sync_copy(hC.at[pl.ds(s * 10240, 10240)],
                  histC.at[c, pl.ds(s * 10240, 10240)])
  pltpu.sync_copy(hD.at[pl.ds(s * 10240, 10240)],
                  histD.at[c, pl.ds(s * 10240, 10240)])


def _run_pass_a(src0, dst0, src1, dst1, h0p, h1p):
  zflat = jnp.zeros((8192,), jnp.float32)
  zrows = jnp.zeros((640, D), jnp.float32)
  ones = jnp.ones((128,), jnp.float32)
  out0 = [
      jax.ShapeDtypeStruct((NC, N0P), jnp.float32),
      jax.ShapeDtypeStruct((NC, N0P), jnp.float32),
      jax.ShapeDtypeStruct((NC, N0P, D), jnp.float32),
      jax.ShapeDtypeStruct((N1P, D), jnp.float32),
  ]
  scratch0 = [
      pltpu.VMEM_SHARED((N0P,), jnp.float32),
      pltpu.VMEM_SHARED((N0P,), jnp.float32),
      pltpu.VMEM_SHARED((N0P, D), jnp.float32),
      pltpu.VMEM((40, 128), jnp.int32),
      pltpu.VMEM((40, 128), jnp.int32),
      pltpu.VMEM((1, 128), jnp.float32),
      pltpu.VMEM((128, D), jnp.float32),
      pltpu.VMEM((128, D), jnp.float32),
      pltpu.SemaphoreType.DMA,
  ]
  f0 = pl.kernel(_pass_a0_body, out_type=out0, mesh=_mesh(),
                 compiler_params=_SC_PARAMS, scratch_types=scratch0)
  histA, histB, tdp, bu = f0(src0.reshape(E0P // 128, 128),
                             dst0.reshape(E0P // 128, 128),
                             h0p, h1p, zflat, zrows, ones)

  out1 = [
      jax.ShapeDtypeStruct((NC, N1P), jnp.float32),
      jax.ShapeDtypeStruct((NC, N1P), jnp.float32),
  ]
  scratch1 = [
      pltpu.VMEM_SHARED((N1P,), jnp.float32),
      pltpu.VMEM_SHARED((N1P,), jnp.float32),
      pltpu.VMEM((120, 128), jnp.int32),
      pltpu.VMEM((120, 128), jnp.int32),
      pltpu.VMEM((1, 128), jnp.float32),
  ]
  f1 = pl.kernel(_pass_a1_body, out_type=out1, mesh=_mesh(),
                 compiler_params=_SC_PARAMS, scratch_types=scratch1)
  histC, histD = f1(src1.reshape(E1P // 128, 128), dst1.reshape(E1P // 128, 128),
                    zflat, ones)
  return histA, histB, histC, histD, tdp, bu


def _agg0_one(table, out, src0r, dst0r, acc, sbuf, dbuf, gbuf, sem, s):
  pltpu.sync_copy(table.at[pl.ds(s * 640, 640), :], acc.at[pl.ds(s * 640, 640), :])
  pltpu.sync_copy(src0r.at[pl.ds(s * 80, 80), :], sbuf)
  pltpu.sync_copy(dst0r.at[pl.ds(s * 80, 80), :], dbuf)
  plsc.subcore_barrier()

  def body(j, _):
    pltpu.async_copy(table.at[sbuf.at[j]], gbuf, sem).wait()
    pltpu.sync_copy(gbuf, acc.at[dbuf.at[j]], add=True)
    return _

  lax.fori_loop(0, E0P // NS // 128, body, 0)
  plsc.subcore_barrier()
  pltpu.sync_copy(acc.at[pl.ds(s * 640, 640), :], out.at[pl.ds(s * 640, 640), :])


def _agg0_body(p0, q0, src0r, dst0r, aggP, aggQ, acc, sbuf, dbuf, gbuf, sem):
  c = lax.axis_index("c")
  s = lax.axis_index("s")

  @pl.when(c == 0)
  def _():
    _agg0_one(p0, aggP, src0r, dst0r, acc, sbuf, dbuf, gbuf, sem, s)

  @pl.when(c == 1)
  def _():
    _agg0_one(q0, aggQ, src0r, dst0r, acc, sbuf, dbuf, gbuf, sem, s)


def _run_agg0(p0, q0, src0r, dst0r):
  out_type = [
      jax.ShapeDtypeStruct((N0P, D), jnp.float32),
      jax.ShapeDtypeStruct((N0P, D), jnp.float32),
  ]
  scratch = [
      pltpu.VMEM_SHARED((N0P, D), jnp.float32),
      pltpu.VMEM((80, 128), jnp.int32),
      pltpu.VMEM((80, 128), jnp.int32),
      pltpu.VMEM((128, D), jnp.float32),
      pltpu.SemaphoreType.DMA,
  ]
  f = pl.kernel(_agg0_body, out_type=out_type, mesh=_mesh(),
                compiler_params=_SC_PARAMS, scratch_types=scratch)
  return f(p0, q0, src0r, dst0r)


_IOTA = None


def _extract(vec16a, vec16b, r):
  io = lax.iota(jnp.int32, L)
  va = jnp.sum(jnp.where(io == r, vec16a, 0))
  vb = jnp.sum(jnp.where(io + L == r, vec16b, 0))
  return va + vb


def _agg1_one(table, out, src1r, dst1r, acc, sbuf, dbuf, cbuf, hist, lcur,
              idxgA, idxgB, idxsA, idxsB, gbuf, semA, semS, s):
  io = lax.iota(jnp.int32, L)

  hist[pl.ds(0, L)] = jnp.zeros((L,), jnp.int32)
  hist[pl.ds(L, L)] = jnp.zeros((L,), jnp.int32)

  def h_outer(jb, _):
    pltpu.sync_copy(dst1r.at[pl.ds(s * 240 + jb * 24, 24), :], dbuf)

    def h_in(j, _2):
      for g in range(8):
        d = dbuf[j, pl.ds(g * L, L)]
        bkt = jnp.right_shift(d, 13)
        rank, last = plsc.scan_count(bkt)
        plsc.addupdate_scatter(hist, [bkt], rank, mask=last)
      return _2

    lax.fori_loop(0, 24, h_in, 0)
    return _

  lax.fori_loop(0, 10, h_outer, 0)

  h0v = hist[pl.ds(0, L)]
  h1v = hist[pl.ds(L, L)]
  a0 = jnp.left_shift(jnp.right_shift(h0v + 63, 6), 6)
  a1 = jnp.left_shift(jnp.right_shift(h1v + 63, 6), 6)
  c0 = plsc.cumsum(a0)
  c1v = plsc.cumsum(a1)
  tot0 = jnp.sum(a0)
  lcur[pl.ds(0, L)] = c0 - a0
  lcur[pl.ds(L, L)] = c1v - a1 + tot0
  lstart0 = c0 - a0
  lstart1 = c1v - a1 + tot0

  def p_body(j, _):
    for g in range(8):
      row = (j * 8 + g)
      padS = (jnp.bitwise_and(row * 128 + io * 8, NB - 1)
              + (19 * NB)).astype(jnp.uint32)
      padD = (NB + jnp.bitwise_and(row + io * 8, 127)).astype(jnp.uint32)
      pad = plsc.bitcast(jnp.bitwise_or(padS, jnp.left_shift(padD, 18)),
                         jnp.int32)
      for t in range(8):
        cbuf[row, pl.ds(t * L, L)] = pad
    return _

  lax.fori_loop(0, CROWS // 8, p_body, 0)

  def s_outer(jb, _):
    pltpu.sync_copy(src1r.at[pl.ds(s * 240 + jb * 24, 24), :], sbuf)
    pltpu.sync_copy(dst1r.at[pl.ds(s * 240 + jb * 24, 24), :], dbuf)

    def s_in(j, _2):
      for g in range(8):
        sv = sbuf[j, pl.ds(g * L, L)]
        d = dbuf[j, pl.ds(g * L, L)]
        bkt = jnp.right_shift(d, 13)
        drel = jnp.bitwise_and(d, NB - 1)
        packed = plsc.bitcast(
            jnp.bitwise_or(sv.astype(jnp.uint32),
                           jnp.left_shift(drel.astype(jnp.uint32), 18)),
            jnp.int32)
        rank, last = plsc.scan_count(bkt)
        basep = plsc.load_gather(lcur, [bkt])
        pos = basep + rank - 1
        hi = jnp.right_shift(pos, 7)
        lo = jnp.bitwise_and(pos, 127)
        plsc.store_scatter(cbuf, [hi, lo], packed)
        plsc.store_scatter(lcur, [bkt], pos + 1, mask=last)
      return _2

    lax.fori_loop(0, 24, s_in, 0)
    return _

  lax.fori_loop(0, 10, s_outer, 0)

  def _drain_one():
    pltpu.make_async_copy(table.at[pl.ds(0, 64), :], gbuf.at[0], semS).wait()

  def r_body(r, _):
    pltpu.sync_copy(table.at[pl.ds(r * NB + s * 512, 512), :],
                    acc.at[pl.ds(s * 512, 512), :])
    plsc.subcore_barrier()
    cnt = _extract(h0v, h1v, r)
    start = _extract(lstart0, lstart1, r)
    blk0 = jnp.right_shift(start, 6)
    nseg = jnp.right_shift(cnt + 63, 6)

    def seg_body(j, _2):
      slot = jnp.bitwise_and(j, 1)
      ab = blk0 + j
      row = jnp.right_shift(ab, 1)
      parity = jnp.bitwise_and(ab, 1)

      @pl.when(j >= 2)
      def _w():
        _drain_one()

      def _unpack(off, idxg_r, idxs_r):
        for t in range(4):
          packed = plsc.bitcast(cbuf[row, pl.ds(off + t * L, L)], jnp.uint32)
          idxg_r[0, pl.ds(t * L, L)] = jnp.bitwise_and(
              packed, jnp.uint32(0x3FFFF)).astype(jnp.int32)
          idxs_r[0, pl.ds(t * L, L)] = jnp.right_shift(
              packed, 18).astype(jnp.int32)

      def _do(idxg_r, idxs_r, gslot):
        @pl.when(parity == 0)
        def _u0():
          _unpack(0, idxg_r, idxs_r)

        @pl.when(parity == 1)
        def _u1():
          _unpack(64, idxg_r, idxs_r)

        pltpu.async_copy(table.at[idxg_r.at[0]], gbuf.at[gslot], semA).wait()
        pltpu.async_copy(gbuf.at[gslot], acc.at[idxs_r.at[0]], semS, add=True)

      @pl.when(slot == 0)
      def _s0():
        _do(idxgA, idxsA, 0)

      @pl.when(slot == 1)
      def _s1():
        _do(idxgB, idxsB, 1)

      return _2

    lax.fori_loop(0, nseg, seg_body, 0)

    def d_body(j, _2):
      _drain_one()
      return _2

    lax.fori_loop(0, jnp.minimum(nseg, 2), d_body, 0)
    plsc.subcore_barrier()
    pltpu.sync_copy(acc.at[pl.ds(s * 512, 512), :],
                    out.at[pl.ds(r * NB + s * 512, 512), :])
    plsc.subcore_barrier()
    return _

  lax.fori_loop(0, NBK, r_body, 0)


def _agg1_body(p1, q1, src1r, dst1r, aggP, aggQ, acc, sbuf, dbuf, cbuf,
               hist, lcur, idxgA, idxgB, idxsA, idxsB, gbuf, semA, semS):
  c = lax.axis_index("c")
  s = lax.axis_index("s")

  @pl.when(c == 0)
  def _():
    _agg1_one(p1, aggP, src1r, dst1r, acc, sbuf, dbuf, cbuf, hist, lcur,
              idxgA, idxgB, idxsA, idxsB, gbuf, semA, semS, s)

  @pl.when(c == 1)
  def _():
    _agg1_one(q1, aggQ, src1r, dst1r, acc, sbuf, dbuf, cbuf, hist, lcur,
              idxgA, idxgB, idxsA, idxsB, gbuf, semA, semS, s)


def _run_agg1(p1, q1, src1r, dst1r):
  out_type = [
      jax.ShapeDtypeStruct((N1P, D), jnp.float32),
      jax.ShapeDtypeStruct((N1P, D), jnp.float32),
  ]
  scratch = [
      pltpu.VMEM_SHARED((NB + 128, D), jnp.float32),
      pltpu.VMEM((24, 128), jnp.int32),
      pltpu.VMEM((24, 128), jnp.int32),
      pltpu.VMEM((CROWS, 128), jnp.int32),
      pltpu.VMEM((2 * L,), jnp.int32),
      pltpu.VMEM((2 * L,), jnp.int32),
      pltpu.VMEM((1, 64), jnp.int32),
      pltpu.VMEM((1, 64), jnp.int32),
      pltpu.VMEM((1, 64), jnp.int32),
      pltpu.VMEM((1, 64), jnp.int32),
      pltpu.VMEM((2, 64, D), jnp.float32),
      pltpu.SemaphoreType.DMA,
      pltpu.SemaphoreType.DMA,
  ]
  f = pl.kernel(_agg1_body, out_type=out_type, mesh=_mesh(),
                compiler_params=_SC_PARAMS, scratch_types=scratch)
  return f(p1, q1, src1r, dst1r)


def _b_level0_kern(h_ref, td0_ref, td1_ref, hA0_ref, hA1_ref, p_ref, q_ref):
  ns = lax.rsqrt(hA0_ref[...] + hA1_ref[...] + 1.0)
  p_ref[...] = h_ref[...] * ns
  q_ref[...] = (td0_ref[...] + td1_ref[...]) * ns


def _b_level1_kern(h_ref, bu_ref, hC0_ref, hC1_ref, p_ref, q_ref):
  ns = lax.rsqrt(hC0_ref[...] + hC1_ref[...] + 1.0)
  p_ref[...] = h_ref[...] * ns
  q_ref[...] = bu_ref[...] * ns


def _run_b(kern, h, fus_args, hh0, hh1, n):
  nb = n // BLK
  row = pl.BlockSpec((BLK, D), lambda i: (i, 0))
  col = pl.BlockSpec((BLK, 1), lambda i: (i, 0))
  in_specs = [row] + [row] * len(fus_args) + [col, col]
  return pl.pallas_call(
      kern,
      grid=(nb,),
      in_specs=in_specs,
      out_specs=[row, row],
      out_shape=[jax.ShapeDtypeStruct((n, D), jnp.float32),
                 jax.ShapeDtypeStruct((n, D), jnp.float32)],
  )(h, *fus_args, hh0, hh1)


def _c_kern(aggP_ref, aggQ_ref, hd0_ref, hd1_ref, Wc_ref, bc_ref, Wf_ref,
            bf_ref, Wr_ref, Wl_ref, bcat_ref, g_ref, b_ref, out_ref):
  nd = lax.rsqrt(hd0_ref[...] + hd1_ref[...] + 1.0)
  cs = jnp.dot(aggP_ref[...] * nd, Wc_ref[...],
               preferred_element_type=jnp.float32) + bc_ref[...]
  fs = jnp.dot(aggQ_ref[...] * nd, Wf_ref[...],
               preferred_element_type=jnp.float32) + bf_ref[...]
  r1 = jnp.maximum(cs, 0.0) + jnp.maximum(fs, 0.0)
  r2 = cs + fs
  res = (jnp.dot(r1, Wr_ref[...], preferred_element_type=jnp.float32)
         + jnp.dot(r2, Wl_ref[...], preferred_element_type=jnp.float32)
         + bcat_ref[...])
  mu = jnp.mean(res, axis=-1, keepdims=True)
  var = jnp.mean(jnp.square(res - mu), axis=-1, keepdims=True)
  out_ref[...] = (res - mu) * lax.rsqrt(var + 1e-5) * g_ref[...] + b_ref[...]


def _run_c(aggP, aggQ, hd0, hd1, Wc, bc, Wf, bf, Wr, Wl, bcat, g, b, n):
  nb = n // BLK
  row = pl.BlockSpec((BLK, D), lambda i: (i, 0))
  col = pl.BlockSpec((BLK, 1), lambda i: (i, 0))
  wmat = pl.BlockSpec((D, D), lambda i: (0, 0))
  wrow = pl.BlockSpec((1, D), lambda i: (0, 0))
  return pl.pallas_call(
      _c_kern,
      grid=(nb,),
      in_specs=[row, row, col, col, wmat, wrow, wmat, wrow, wmat, wmat,
                wrow, wrow, wrow],
      out_specs=row,
      out_shape=jax.ShapeDtypeStruct((n, D), jnp.float32),
  )(aggP, aggQ, hd0, hd1, Wc, bc, Wf, bf, Wr, Wl, bcat, g, b)


def _pad_edges(src, dst, e, ep, n, npad):
  npad_lo = n
  span = npad - n
  i = jnp.arange(ep - e, dtype=jnp.int32)
  fill = npad_lo + (i % span)
  srcp = jnp.concatenate([src, fill])
  dstp = jnp.concatenate([dst, fill])
  return srcp, dstp


def kernel(h0, h1, edge_index0, edge_index1,
           W_conv_td, b_conv_td, W_fus_td, b_fus_td, conv_w_td, td_w,
           Wcat_td, bcat_td, ln_g_td, ln_b_td,
           W_conv_bu, b_conv_bu, W_fus_bu, b_fus_bu, conv_w_bu, bu_w,
           Wcat_bu, bcat_bu, ln_g_bu, ln_b_bu):
  f32 = jnp.float32
  h0p = jnp.zeros((N0P, D), f32).at[:N0].set(h0)
  h1p = jnp.zeros((N1P, D), f32).at[:N1].set(h1)
  src0, dst0 = edge_index0[0], edge_index0[1]
  src1, dst1 = edge_index1[0], edge_index1[1]
  src0p, dst0p = _pad_edges(src0, dst0, E0, E0P, N0, N0P)
  src1p, dst1p = _pad_edges(src1, dst1, E1, E1P, N1, N1P)

  histA, histB, histC, histD, tdp, bu = _run_pass_a(
      src0p, dst0p, src1p, dst1p, h0p, h1p)

  hA0 = histA[0][:, None]
  hA1 = histA[1][:, None]
  hB0 = histB[0][:, None]
  hB1 = histB[1][:, None]
  hC0 = histC[0][:, None]
  hC1 = histC[1][:, None]
  hD0 = histD[0][:, None]
  hD1 = histD[1][:, None]

  p0, q0 = _run_b(_b_level0_kern, h0p, [tdp[0], tdp[1]], hA0, hA1, N0P)
  p1, q1 = _run_b(_b_level1_kern, h1p, [bu], hC0, hC1, N1P)

  aggP0, aggQ0 = _run_agg0(p0, q0, src0p.reshape(E0P // 128, 128),
                           dst0p.reshape(E0P // 128, 128))
  aggP1, aggQ1 = _run_agg1(p1, q1, src1p.reshape(E1P // 128, 128),
                           dst1p.reshape(E1P // 128, 128))

  Wc_td = W_conv_td * conv_w_td[None, :]
  bc_td = (b_conv_td * conv_w_td)[None, :]
  Wf_td = W_fus_td * td_w[None, :]
  bf_td = (b_fus_td * td_w)[None, :]
  Wc_bu = W_conv_bu * conv_w_bu[None, :]
  bc_bu = (b_conv_bu * conv_w_bu)[None, :]
  Wf_bu = W_fus_bu * bu_w[None, :]
  bf_bu = (b_fus_bu * bu_w)[None, :]

  new_h0 = _run_c(aggP0, aggQ0, hB0, hB1, Wc_td, bc_td, Wf_td, bf_td,
                  Wcat_td[:D], Wcat_td[D:], bcat_td[None, :],
                  ln_g_td[None, :], ln_b_td[None, :], N0P)
  new_h1 = _run_c(aggP1, aggQ1, hD0, hD1, Wc_bu, bc_bu, Wf_bu, bf_bu,
                  Wcat_bu[:D], Wcat_bu[D:], bcat_bu[None, :],
                  ln_g_bu[None, :], ln_b_bu[None, :], N1P)

  return (new_h0[:N0], new_h1[:N1])
